# baseline (speedup 1.0000x reference)
import functools

import jax
import jax.numpy as jnp
from jax import lax
from jax.experimental import pallas as pl
from jax.experimental.pallas import tpu as pltpu
from jax.experimental.pallas import tpu_sc as plsc

B = 16384
D = 64
NR = 1000
NC = 2
NS = 16
NW = NC * NS
BPW = B // NW
L = 16

_mesh = plsc.VectorSubcoreMesh(
    core_axis_name="c", subcore_axis_name="s", num_cores=NC, num_subcores=NS
)


def _entity_lookup(idx_hbm, tab, out, base, idxb, rows, sem):
    pltpu.sync_copy(idx_hbm.at[pl.ds(base, BPW)], idxb)

    def group(g, carry):
        svec = idxb[pl.ds(g * L, L)]
        for r in range(L):
            i = svec[r]
            pltpu.async_copy(
                tab.at[pl.ds(i, 1)],
                rows.at[pl.ds(g * L + r, 1)],
                sem,
            )
        return carry

    lax.fori_loop(0, BPW // L, group, 0, unroll=False)
    pltpu.make_async_copy(tab.at[pl.ds(0, BPW)], rows, sem).wait()
    pltpu.sync_copy(rows, out.at[pl.ds(base, BPW)])


@functools.partial(
    pl.kernel,
    out_type=(
        jax.ShapeDtypeStruct((B, D), jnp.float32),
        jax.ShapeDtypeStruct((B, D), jnp.float32),
    ),
    mesh=_mesh,
    scratch_types=[
        pltpu.VMEM((BPW,), jnp.int32),
        pltpu.VMEM((BPW, D), jnp.float32),
        pltpu.SemaphoreType.DMA,
    ],
)
def _sc_entity(head_hbm, tail_hbm, etab,
               out_h, out_t, idxb, rows, sem):
    wid = lax.axis_index("s") * NC + lax.axis_index("c")
    base = wid * BPW
    _entity_lookup(head_hbm, etab, out_h, base, idxb, rows, sem)
    _entity_lookup(tail_hbm, etab, out_t, base, idxb, rows, sem)


@functools.partial(
    pl.kernel,
    out_type=jax.ShapeDtypeStruct((B, D), jnp.float32),
    mesh=_mesh,
    compiler_params=pltpu.CompilerParams(needs_layout_passes=False),
    scratch_types=[
        pltpu.VMEM((BPW,), jnp.int32),
        pltpu.VMEM((256, D), jnp.float32),
        pltpu.VMEM((BPW, D), jnp.float32),
        pltpu.SemaphoreType.DMA,
    ],
)
def _sc_relation(rel_hbm, rtab, out_r, idxb, relv, rows, sem):
    wid = lax.axis_index("s") * NC + lax.axis_index("c")
    base = wid * BPW
    pltpu.sync_copy(rel_hbm.at[pl.ds(base, BPW)], idxb)
    for p, lo in enumerate((0, 256, 512, NR - 256)):
        pltpu.async_copy(rtab.at[pl.ds(lo, 256)], relv, sem).wait()

        def group(g, carry, p=p, lo=lo):
            svec = idxb[pl.ds(g * L, L)]
            lvec = jnp.minimum(jnp.maximum(svec - lo, 0), 255)
            mvec = ((svec >= lo) & (svec < lo + 256)).astype(jnp.int32)
            lanes = lax.iota(jnp.int32, L)
            for r in range(L):
                i = lvec[r]
                rsplat = jnp.full((L,), i, jnp.int32)
                mb = jnp.full((L,), mvec[r], jnp.int32)
                for cc in range(D // L):
                    sl = pl.ds(cc * L, L)
                    picked = plsc.load_gather(relv, [rsplat, lanes + cc * L])
                    if p == 0:
                        rows[g * L + r, sl] = picked
                    else:
                        rows[g * L + r, sl] = jnp.where(
                            mb == 1, picked, rows[g * L + r, sl]
                        )
            return carry

        lax.fori_loop(0, BPW // L, group, 0, unroll=False)
    pltpu.sync_copy(rows, out_r.at[pl.ds(base, BPW)])


def kernel(head, relation, tail, entity_table, relation_table):
    h = head.astype(jnp.int32)
    r = relation.astype(jnp.int32)
    t = tail.astype(jnp.int32)
    out_h, out_t = _sc_entity(h, t, entity_table)
    out_r = _sc_relation(r, relation_table)
    return (out_h, out_r, out_t)


# speedup vs baseline: 1.0842x; 1.0842x over previous
import functools

import jax
import jax.numpy as jnp
from jax import lax
from jax.experimental import pallas as pl
from jax.experimental.pallas import tpu as pltpu
from jax.experimental.pallas import tpu_sc as plsc

B = 16384
D = 64
NC = 2
NS = 16
NW = NC * NS
BPW = B // NW
G = 16

_mesh = plsc.VectorSubcoreMesh(
    core_axis_name="c", subcore_axis_name="s", num_cores=NC, num_subcores=NS
)


def _lookup(idx_hbm, tab, out, base, idxb, rows, sem):
    pltpu.sync_copy(idx_hbm.at[pl.ds(base, BPW)], idxb)

    def group(g, carry):
        svec = idxb[pl.ds(g * G, G)]
        for r in range(G):
            i = svec[r]
            pltpu.async_copy(
                tab.at[pl.ds(i, 1)],
                rows.at[pl.ds(g * G + r, 1)],
                sem,
            )
        return carry

    lax.fori_loop(0, BPW // G, group, 0, unroll=False)
    pltpu.make_async_copy(tab.at[pl.ds(0, BPW)], rows, sem).wait()
    pltpu.sync_copy(rows, out.at[pl.ds(base, BPW)])


@functools.partial(
    pl.kernel,
    out_type=(
        jax.ShapeDtypeStruct((B, D), jnp.float32),
        jax.ShapeDtypeStruct((B, D), jnp.float32),
        jax.ShapeDtypeStruct((B, D), jnp.float32),
    ),
    mesh=_mesh,
    scratch_types=[
        pltpu.VMEM((BPW,), jnp.int32),
        pltpu.VMEM((BPW, D), jnp.float32),
        pltpu.SemaphoreType.DMA,
    ],
)
def _sc_gather(head_hbm, rel_hbm, tail_hbm, etab, rtab,
               out_h, out_r, out_t,
               idxb, rows, sem):
    wid = lax.axis_index("s") * NC + lax.axis_index("c")
    base = wid * BPW
    _lookup(head_hbm, etab, out_h, base, idxb, rows, sem)
    _lookup(tail_hbm, etab, out_t, base, idxb, rows, sem)
    _lookup(rel_hbm, rtab, out_r, base, idxb, rows, sem)


def kernel(head, relation, tail, entity_table, relation_table):
    h = head.astype(jnp.int32)
    r = relation.astype(jnp.int32)
    t = tail.astype(jnp.int32)
    return _sc_gather(h, r, t, entity_table, relation_table)
